# Initial kernel scaffold; baseline (speedup 1.0000x reference)
import jax
import jax.numpy as jnp
from jax.experimental import pallas as pl


def kernel(x, router_kernel, router_bias, expert_kernels, expert_biases, out_kernel, out_bias):
    raise NotImplementedError("write your pallas kernel here")


# speedup vs baseline: 1.2992x; 1.2992x over previous
import functools

import jax
import jax.numpy as jnp
from jax.experimental import pallas as pl
from jax.experimental.pallas import tpu as pltpu

_E = 8
_D = 1024
_H = 1024
_T = 512
_N = 4096


def _moe_dense_kernel(x_ref, rk_ref, rb_ref, ek_ref, eb_ref, wo_ref, ob_ref,
                      out_ref):
    x = x_ref[...]
    logits = jnp.dot(x, rk_ref[...], preferred_element_type=jnp.float32)
    logits = logits + rb_ref[...]
    m = jnp.max(logits, axis=-1, keepdims=True)
    ex = jnp.exp(logits - m)
    probs = ex / jnp.sum(ex, axis=-1, keepdims=True)

    lane = jax.lax.broadcasted_iota(jnp.int32, probs.shape, 1)
    p1 = jnp.max(probs, axis=-1, keepdims=True)
    i1 = jnp.min(jnp.where(probs == p1, lane, _E), axis=-1, keepdims=True)
    probs2 = jnp.where(lane == i1, -jnp.inf, probs)
    p2 = jnp.max(probs2, axis=-1, keepdims=True)
    i2 = jnp.min(jnp.where(probs2 == p2, lane, _E), axis=-1, keepdims=True)
    denom = p1 + p2
    g1 = p1 / denom
    g2 = p2 / denom

    acc = jnp.zeros((x.shape[0], _H), jnp.float32)
    for e in range(_E):
        h = jnp.dot(x, ek_ref[e], preferred_element_type=jnp.float32)
        h = h + eb_ref[e][None, :]
        h = jax.nn.gelu(h)
        gate = (jnp.where(i1 == e, g1, 0.0) + jnp.where(i2 == e, g2, 0.0))
        acc = acc + h * gate
    out = jnp.dot(acc.astype(jnp.bfloat16), wo_ref[...],
                  preferred_element_type=jnp.float32)
    out_ref[...] = out + ob_ref[...]


@functools.partial(jax.jit, static_argnames=())
def kernel(x, router_kernel, router_bias, expert_kernels, expert_biases,
           out_kernel, out_bias):
    b, s, d = x.shape
    xf = x.reshape(b * s, d).astype(jnp.bfloat16)
    rk = router_kernel.astype(jnp.bfloat16)
    ek = expert_kernels.astype(jnp.bfloat16)
    wo = out_kernel.astype(jnp.bfloat16)
    rb = router_bias.reshape(1, _E)
    ob = out_bias.reshape(1, _H)

    grid = (_N // _T,)
    out = pl.pallas_call(
        _moe_dense_kernel,
        grid=grid,
        in_specs=[
            pl.BlockSpec((_T, _D), lambda i: (i, 0)),
            pl.BlockSpec((_D, _E), lambda i: (0, 0)),
            pl.BlockSpec((1, _E), lambda i: (0, 0)),
            pl.BlockSpec((_E, _D, _H), lambda i: (0, 0, 0)),
            pl.BlockSpec((_E, _H), lambda i: (0, 0)),
            pl.BlockSpec((_H, _H), lambda i: (0, 0)),
            pl.BlockSpec((1, _H), lambda i: (0, 0)),
        ],
        out_specs=pl.BlockSpec((_T, _H), lambda i: (i, 0)),
        out_shape=jax.ShapeDtypeStruct((_N, _H), jnp.float32),
        compiler_params=pltpu.CompilerParams(
            dimension_semantics=("arbitrary",),
        ),
    )(xf, rk, rb, ek, expert_biases, wo, ob)
    return out.reshape(b, s, d)
